# Initial kernel scaffold; baseline (speedup 1.0000x reference)
import jax
import jax.numpy as jnp
from jax.experimental import pallas as pl


def kernel(x_author, x_paper, edge_index_writes, edge_index_rev_writes, params):
    raise NotImplementedError("write your pallas kernel here")


# speedup vs baseline: 8.1212x; 8.1212x over previous
import functools
import jax
import jax.numpy as jnp
from jax import lax
from jax.experimental import pallas as pl
from jax.experimental.pallas import tpu as pltpu
from jax.experimental.pallas import tpu_sc as plsc

N = 10000
NP = 10112
HID = 512
H = 8
D = 64
E = 160000
EP = 163840
NC, NS = 2, 16
NW = NC * NS
TPW = EP // NW
CH = 128
NCH = TPW // CH
MW = 80
RSL = NP // NS

_mesh = plsc.VectorSubcoreMesh(
    core_axis_name="c", subcore_axis_name="s", num_cores=NC, num_subcores=NS
)


def _linear(x, w, b, act, br=1000):
    m, k = x.shape
    n = w.shape[1]

    def body(x_ref, w_ref, b_ref, o_ref):
        y = jnp.dot(x_ref[...], w_ref[...], preferred_element_type=jnp.float32)
        y = y + b_ref[...]
        o_ref[...] = act(y)

    return pl.pallas_call(
        body,
        grid=(m // br,),
        in_specs=[
            pl.BlockSpec((br, k), lambda i: (i, 0)),
            pl.BlockSpec((k, n), lambda i: (0, 0)),
            pl.BlockSpec((1, n), lambda i: (0, 0)),
        ],
        out_specs=pl.BlockSpec((br, n), lambda i: (i, 0)),
        out_shape=jax.ShapeDtypeStruct((m, n), jnp.float32),
    )(x, w, b.reshape(1, n))


def _block_diag(rel):
    bd = jnp.zeros((HID, HID), jnp.float32)
    for h in range(H):
        bd = bd.at[h * D:(h + 1) * D, h * D:(h + 1) * D].set(rel[h])
    return bd


def _edge_messages(qe, ke, ve, prel128, be=2048):
    def body(q_ref, k_ref, v_ref, p_ref, o_ref):
        i = pl.program_id(0)
        q = q_ref[...]
        k = k_ref[...]
        v = v_ref[...]
        rows = i * be + lax.broadcasted_iota(jnp.int32, (be, 1), 0)
        mask = rows < E
        lane = lax.broadcasted_iota(jnp.int32, (be, 16), 1)
        for h in range(H):
            sl = slice(h * D, (h + 1) * D)
            ah = jnp.sum(q[:, sl] * k[:, sl], axis=1, keepdims=True)
            ah = ah * (p_ref[0, h] * 0.125)
            exh = jnp.where(mask, jnp.exp(ah), 0.0)
            o_ref[h] = jnp.concatenate(
                [v[:, sl] * exh, jnp.where(lane == 0, exh, 0.0)], axis=1
            )

    return pl.pallas_call(
        body,
        grid=(EP // be,),
        in_specs=[
            pl.BlockSpec((be, HID), lambda i: (i, 0)),
            pl.BlockSpec((be, HID), lambda i: (i, 0)),
            pl.BlockSpec((be, HID), lambda i: (i, 0)),
            pl.BlockSpec((1, 128), lambda i: (0, 0)),
        ],
        out_specs=pl.BlockSpec((H, be, MW), lambda i: (0, i, 0)),
        out_shape=jax.ShapeDtypeStruct((H, EP, MW), jnp.float32),
    )(qe, ke, ve, prel128)


def _normalize(acc, br=1000):
    def body(a_ref, o_ref):
        a = a_ref[0] + a_ref[1]
        pieces = [
            a[h, :, 0:D] / (a[h, :, D:D + 1] + 1e-16) for h in range(H)
        ]
        o_ref[...] = jnp.concatenate(pieces, axis=1)

    return pl.pallas_call(
        body,
        grid=(N // br,),
        in_specs=[
            pl.BlockSpec((2, H, br, MW), lambda i: (0, 0, i, 0)),
        ],
        out_specs=pl.BlockSpec((br, HID), lambda i: (i, 0)),
        out_shape=jax.ShapeDtypeStruct((N, HID), jnp.float32),
    )(acc)


def _out_proj(msgn, wa, ba, skip128, xprev, br=1000):
    def body(m_ref, w_ref, b_ref, s_ref, x_ref, o_ref):
        o = jnp.dot(jax.nn.gelu(m_ref[...]), w_ref[...],
                    preferred_element_type=jnp.float32) + b_ref[...]
        beta = jax.nn.sigmoid(s_ref[0, 0])
        o_ref[...] = beta * o + (1.0 - beta) * x_ref[...]

    return pl.pallas_call(
        body,
        grid=(N // br,),
        in_specs=[
            pl.BlockSpec((br, HID), lambda i: (i, 0)),
            pl.BlockSpec((HID, HID), lambda i: (0, 0)),
            pl.BlockSpec((1, HID), lambda i: (0, 0)),
            pl.BlockSpec((1, 128), lambda i: (0, 0)),
            pl.BlockSpec((br, HID), lambda i: (i, 0)),
        ],
        out_specs=pl.BlockSpec((br, HID), lambda i: (i, 0)),
        out_shape=jax.ShapeDtypeStruct((N, HID), jnp.float32),
    )(msgn, wa, ba.reshape(1, HID), skip128, xprev)


@functools.partial(
    pl.kernel,
    mesh=_mesh,
    out_type=(
        jax.ShapeDtypeStruct((EP, HID), jnp.float32),
        jax.ShapeDtypeStruct((EP, HID), jnp.float32),
        jax.ShapeDtypeStruct((EP, HID), jnp.float32),
    ),
    scratch_types=[
        pltpu.VMEM((NCH, CH), jnp.int32),
        pltpu.VMEM((NCH, CH), jnp.int32),
        pltpu.VMEM((CH, HID), jnp.float32),
        pltpu.SemaphoreType.DMA,
    ],
)
def _sc_gather(qtab, ktab, vtab, didx_hbm, sidx_hbm,
               qe, ke, ve, didx_v, sidx_v, rows_v, sem):
    c = lax.axis_index("c")
    s = lax.axis_index("s")
    wid = s * NC + c
    base = wid * TPW
    pltpu.sync_copy(didx_hbm.at[wid], didx_v)
    pltpu.sync_copy(sidx_hbm.at[wid], sidx_v)

    def chunk(j, carry):
        off = pl.multiple_of(base + j * CH, CH)
        pltpu.async_copy(qtab.at[didx_v.at[j]], rows_v, sem).wait()
        pltpu.sync_copy(rows_v, qe.at[pl.ds(off, CH)])
        pltpu.async_copy(ktab.at[sidx_v.at[j]], rows_v, sem).wait()
        pltpu.sync_copy(rows_v, ke.at[pl.ds(off, CH)])
        pltpu.async_copy(vtab.at[sidx_v.at[j]], rows_v, sem).wait()
        pltpu.sync_copy(rows_v, ve.at[pl.ds(off, CH)])
        return carry

    lax.fori_loop(0, NCH, chunk, 0)


@functools.partial(
    pl.kernel,
    mesh=_mesh,
    out_type=jax.ShapeDtypeStruct((NC, H, NP, MW), jnp.float32),
    scratch_types=[
        pltpu.VMEM((NCH, CH), jnp.int32),
        pltpu.VMEM((CH, MW), jnp.float32),
        pltpu.VMEM_SHARED((NP, MW), jnp.float32),
        pltpu.SemaphoreType.DMA,
    ],
)
def _sc_scatter(msg_hbm, didx_hbm, zeros_hbm, out_hbm,
                didx_v, rows_v, acc_sh, sem):
    c = lax.axis_index("c")
    s = lax.axis_index("s")
    wid = s * NC + c
    base = wid * TPW
    pltpu.sync_copy(didx_hbm.at[wid], didx_v)
    myrows = pl.ds(s * RSL, RSL)
    for h in range(H):
        pltpu.sync_copy(zeros_hbm.at[myrows], acc_sh.at[myrows])
        plsc.subcore_barrier()

        def chunk(j, carry):
            off = pl.multiple_of(base + j * CH, CH)
            pltpu.sync_copy(msg_hbm.at[h].at[pl.ds(off, CH)], rows_v)
            pltpu.sync_copy(rows_v, acc_sh.at[didx_v.at[j]], add=True)
            return carry

        lax.fori_loop(0, NCH, chunk, 0)
        plsc.subcore_barrier()
        pltpu.sync_copy(acc_sh.at[myrows], out_hbm.at[c].at[h].at[myrows])
        plsc.subcore_barrier()


def _prep_idx(ei):
    s = jnp.zeros((EP,), jnp.int32).at[:E].set(ei[0])
    d = jnp.zeros((EP,), jnp.int32).at[:E].set(ei[1])
    return d.reshape(NW, NCH, CH), s.reshape(NW, NCH, CH)


def _pad128(v):
    v = v.reshape(-1)
    return jnp.zeros((1, 128), jnp.float32).at[0, : v.shape[0]].set(v)


def kernel(x_author, x_paper, edge_index_writes, edge_index_rev_writes, params):
    p = params
    zeros_acc = jnp.zeros((NP, MW), jnp.float32)
    idx = {
        "writes": _prep_idx(edge_index_writes),
        "rev_writes": _prep_idx(edge_index_rev_writes),
    }
    edge_types = [
        ("author", "writes", "paper"),
        ("paper", "rev_writes", "author"),
    ]

    x = {
        "author": _linear(x_author, p["Win_author"], p["bin_author"], jax.nn.relu),
        "paper": _linear(x_paper, p["Win_paper"], p["bin_paper"], jax.nn.relu),
    }

    for l in range(2):
        q, k, v = {}, {}, {}
        for nt in ("author", "paper"):
            wqkv = jnp.concatenate(
                [p[f"Wq_{l}_{nt}"], p[f"Wk_{l}_{nt}"], p[f"Wv_{l}_{nt}"]], axis=1
            )
            bqkv = jnp.concatenate(
                [p[f"bWq_{l}_{nt}"], p[f"bWk_{l}_{nt}"], p[f"bWv_{l}_{nt}"]]
            )
            qkv = _linear(x[nt], wqkv, bqkv, lambda y: y)
            q[nt] = qkv[:, :HID]
            k[nt] = qkv[:, HID:2 * HID]
            v[nt] = qkv[:, 2 * HID:]

        msgn = {}
        zb = jnp.zeros((HID,), jnp.float32)
        for (src, rel, dst) in edge_types:
            krel = _linear(k[src], _block_diag(p[f"arel_{l}_{rel}"]), zb, lambda y: y)
            vrel = _linear(v[src], _block_diag(p[f"mrel_{l}_{rel}"]), zb, lambda y: y)
            didx3, sidx3 = idx[rel]
            qe, ke, ve = _sc_gather(q[dst], krel, vrel, didx3, sidx3)
            msgx = _edge_messages(qe, ke, ve, _pad128(p[f"prel_{l}_{rel}"]))
            acc = _sc_scatter(msgx, didx3, zeros_acc)
            msgn[dst] = _normalize(acc)

        for nt in ("author", "paper"):
            x[nt] = _out_proj(
                msgn[nt],
                p[f"Wa_{l}_{nt}"],
                p[f"bWa_{l}_{nt}"],
                _pad128(p[f"skip_{l}_{nt}"]),
                x[nt],
            )

    return x["author"], x["paper"]


# speedup vs baseline: 9.6812x; 1.1921x over previous
import functools
import jax
import jax.numpy as jnp
from jax import lax
from jax.experimental import pallas as pl
from jax.experimental.pallas import tpu as pltpu
from jax.experimental.pallas import tpu_sc as plsc

N = 10000
NP = 10112
HID = 512
H = 8
D = 64
E = 160000
EP = 163840
NC, NS = 2, 16
NW = NC * NS
TPW = EP // NW
CH = 128
NCH = TPW // CH
CH2 = 64
NCH2 = TPW // CH2
MW = 80
RSL = NP // NS

_mesh = plsc.VectorSubcoreMesh(
    core_axis_name="c", subcore_axis_name="s", num_cores=NC, num_subcores=NS
)


def _linear(x, w, b, act, br=1000):
    m, k = x.shape
    n = w.shape[1]

    def body(x_ref, w_ref, b_ref, o_ref):
        y = jnp.dot(x_ref[...], w_ref[...], preferred_element_type=jnp.float32)
        y = y + b_ref[...]
        o_ref[...] = act(y)

    return pl.pallas_call(
        body,
        grid=(m // br,),
        in_specs=[
            pl.BlockSpec((br, k), lambda i: (i, 0)),
            pl.BlockSpec((k, n), lambda i: (0, 0)),
            pl.BlockSpec((1, n), lambda i: (0, 0)),
        ],
        out_specs=pl.BlockSpec((br, n), lambda i: (i, 0)),
        out_shape=jax.ShapeDtypeStruct((m, n), jnp.float32),
    )(x, w, b.reshape(1, n))


def _block_diag(rel):
    bd = jnp.zeros((HID, HID), jnp.float32)
    for h in range(H):
        bd = bd.at[h * D:(h + 1) * D, h * D:(h + 1) * D].set(rel[h])
    return bd


def _kv_rel(k, v, bda, bdm, br=1000):
    def body(k_ref, v_ref, a_ref, m_ref, o_ref):
        kr = jnp.dot(k_ref[...], a_ref[...], preferred_element_type=jnp.float32)
        vr = jnp.dot(v_ref[...], m_ref[...], preferred_element_type=jnp.float32)
        o_ref[...] = jnp.concatenate([kr, vr], axis=1)

    return pl.pallas_call(
        body,
        grid=(N // br,),
        in_specs=[
            pl.BlockSpec((br, HID), lambda i: (i, 0)),
            pl.BlockSpec((br, HID), lambda i: (i, 0)),
            pl.BlockSpec((HID, HID), lambda i: (0, 0)),
            pl.BlockSpec((HID, HID), lambda i: (0, 0)),
        ],
        out_specs=pl.BlockSpec((br, 2 * HID), lambda i: (i, 0)),
        out_shape=jax.ShapeDtypeStruct((N, 2 * HID), jnp.float32),
    )(k, v, bda, bdm)


def _edge_messages(qe, kve, prel128, be=2048):
    def body(q_ref, kv_ref, p_ref, o_ref):
        i = pl.program_id(0)
        q = q_ref[...]
        k = kv_ref[:, :HID]
        v = kv_ref[:, HID:]
        rows = i * be + lax.broadcasted_iota(jnp.int32, (be, 1), 0)
        mask = rows < E
        lane = lax.broadcasted_iota(jnp.int32, (be, 16), 1)
        for h in range(H):
            sl = slice(h * D, (h + 1) * D)
            ah = jnp.sum(q[:, sl] * k[:, sl], axis=1, keepdims=True)
            ah = ah * (p_ref[0, h] * 0.125)
            exh = jnp.where(mask, jnp.exp(ah), 0.0)
            o_ref[h] = jnp.concatenate(
                [v[:, sl] * exh, jnp.where(lane == 0, exh, 0.0)], axis=1
            )

    return pl.pallas_call(
        body,
        grid=(EP // be,),
        in_specs=[
            pl.BlockSpec((be, HID), lambda i: (i, 0)),
            pl.BlockSpec((be, 2 * HID), lambda i: (i, 0)),
            pl.BlockSpec((1, 128), lambda i: (0, 0)),
        ],
        out_specs=pl.BlockSpec((H, be, MW), lambda i: (0, i, 0)),
        out_shape=jax.ShapeDtypeStruct((H, EP, MW), jnp.float32),
    )(qe, kve, prel128)


def _normalize(acc, br=1000):
    def body(a_ref, o_ref):
        a = a_ref[0] + a_ref[1]
        pieces = [
            a[h, :, 0:D] / (a[h, :, D:D + 1] + 1e-16) for h in range(H)
        ]
        o_ref[...] = jnp.concatenate(pieces, axis=1)

    return pl.pallas_call(
        body,
        grid=(N // br,),
        in_specs=[
            pl.BlockSpec((2, H, br, MW), lambda i: (0, 0, i, 0)),
        ],
        out_specs=pl.BlockSpec((br, HID), lambda i: (i, 0)),
        out_shape=jax.ShapeDtypeStruct((N, HID), jnp.float32),
    )(acc)


def _out_proj(msgn, wa, ba, skip128, xprev, br=1000):
    def body(m_ref, w_ref, b_ref, s_ref, x_ref, o_ref):
        o = jnp.dot(jax.nn.gelu(m_ref[...]), w_ref[...],
                    preferred_element_type=jnp.float32) + b_ref[...]
        beta = jax.nn.sigmoid(s_ref[0, 0])
        o_ref[...] = beta * o + (1.0 - beta) * x_ref[...]

    return pl.pallas_call(
        body,
        grid=(N // br,),
        in_specs=[
            pl.BlockSpec((br, HID), lambda i: (i, 0)),
            pl.BlockSpec((HID, HID), lambda i: (0, 0)),
            pl.BlockSpec((1, HID), lambda i: (0, 0)),
            pl.BlockSpec((1, 128), lambda i: (0, 0)),
            pl.BlockSpec((br, HID), lambda i: (i, 0)),
        ],
        out_specs=pl.BlockSpec((br, HID), lambda i: (i, 0)),
        out_shape=jax.ShapeDtypeStruct((N, HID), jnp.float32),
    )(msgn, wa, ba.reshape(1, HID), skip128, xprev)


@functools.partial(
    pl.kernel,
    mesh=_mesh,
    out_type=(
        jax.ShapeDtypeStruct((EP, HID), jnp.float32),
        jax.ShapeDtypeStruct((EP, 2 * HID), jnp.float32),
    ),
    scratch_types=[
        pltpu.VMEM((NCH2, CH2), jnp.int32),
        pltpu.VMEM((NCH2, CH2), jnp.int32),
        pltpu.VMEM((CH2, HID), jnp.float32),
        pltpu.VMEM((CH2, 2 * HID), jnp.float32),
        pltpu.SemaphoreType.DMA,
        pltpu.SemaphoreType.DMA,
        pltpu.SemaphoreType.DMA,
        pltpu.SemaphoreType.DMA,
    ],
)
def _sc_gather(qtab, kvtab, didx_hbm, sidx_hbm,
               qe, kve, didx_v, sidx_v, bq, bkv, semq, semk, ssq, ssk):
    c = lax.axis_index("c")
    s = lax.axis_index("s")
    wid = s * NC + c
    base = wid * TPW
    pltpu.sync_copy(didx_hbm.at[wid], didx_v)
    pltpu.sync_copy(sidx_hbm.at[wid], sidx_v)

    def chunk(j, carry):
        off = pl.multiple_of(base + j * CH2, CH2)
        gq = pltpu.async_copy(qtab.at[didx_v.at[j]], bq, semq)
        gk = pltpu.async_copy(kvtab.at[sidx_v.at[j]], bkv, semk)
        gq.wait()
        sq = pltpu.async_copy(bq, qe.at[pl.ds(off, CH2)], ssq)
        gk.wait()
        sk = pltpu.async_copy(bkv, kve.at[pl.ds(off, CH2)], ssk)
        sq.wait()
        sk.wait()
        return carry

    lax.fori_loop(0, NCH2, chunk, 0)


@functools.partial(
    pl.kernel,
    mesh=_mesh,
    out_type=jax.ShapeDtypeStruct((NC, H, NP, MW), jnp.float32),
    scratch_types=[
        pltpu.VMEM((NCH, CH), jnp.int32),
        pltpu.VMEM((CH, MW), jnp.float32),
        pltpu.VMEM_SHARED((NP, MW), jnp.float32),
        pltpu.SemaphoreType.DMA,
    ],
)
def _sc_scatter(msg_hbm, didx_hbm, zeros_hbm, out_hbm,
                didx_v, rows_v, acc_sh, sem):
    c = lax.axis_index("c")
    s = lax.axis_index("s")
    wid = s * NC + c
    base = wid * TPW
    pltpu.sync_copy(didx_hbm.at[wid], didx_v)
    myrows = pl.ds(s * RSL, RSL)
    for h in range(H):
        pltpu.sync_copy(zeros_hbm.at[myrows], acc_sh.at[myrows])
        plsc.subcore_barrier()

        def chunk(j, carry):
            off = pl.multiple_of(base + j * CH, CH)
            pltpu.async_copy(
                msg_hbm.at[h].at[pl.ds(off, CH)], rows_v, sem
            ).wait()
            pltpu.sync_copy(rows_v, acc_sh.at[didx_v.at[j]], add=True)
            return carry

        lax.fori_loop(0, NCH, chunk, 0)
        plsc.subcore_barrier()
        pltpu.sync_copy(acc_sh.at[myrows], out_hbm.at[c].at[h].at[myrows])
        plsc.subcore_barrier()


def _prep_idx(ei):
    s = jnp.zeros((EP,), jnp.int32).at[:E].set(ei[0])
    d = jnp.zeros((EP,), jnp.int32).at[:E].set(ei[1])
    return (
        d.reshape(NW, NCH2, CH2),
        s.reshape(NW, NCH2, CH2),
        d.reshape(NW, NCH, CH),
    )


def _pad128(v):
    v = v.reshape(-1)
    return jnp.zeros((1, 128), jnp.float32).at[0, : v.shape[0]].set(v)


def kernel(x_author, x_paper, edge_index_writes, edge_index_rev_writes, params):
    p = params
    zeros_acc = jnp.zeros((NP, MW), jnp.float32)
    idx = {
        "writes": _prep_idx(edge_index_writes),
        "rev_writes": _prep_idx(edge_index_rev_writes),
    }
    edge_types = [
        ("author", "writes", "paper"),
        ("paper", "rev_writes", "author"),
    ]

    x = {
        "author": _linear(x_author, p["Win_author"], p["bin_author"], jax.nn.relu),
        "paper": _linear(x_paper, p["Win_paper"], p["bin_paper"], jax.nn.relu),
    }

    for l in range(2):
        q, k, v = {}, {}, {}
        for nt in ("author", "paper"):
            wqkv = jnp.concatenate(
                [p[f"Wq_{l}_{nt}"], p[f"Wk_{l}_{nt}"], p[f"Wv_{l}_{nt}"]], axis=1
            )
            bqkv = jnp.concatenate(
                [p[f"bWq_{l}_{nt}"], p[f"bWk_{l}_{nt}"], p[f"bWv_{l}_{nt}"]]
            )
            qkv = _linear(x[nt], wqkv, bqkv, lambda y: y)
            q[nt] = qkv[:, :HID]
            k[nt] = qkv[:, HID:2 * HID]
            v[nt] = qkv[:, 2 * HID:]

        msgn = {}
        for (src, rel, dst) in edge_types:
            kvtab = _kv_rel(
                k[src], v[src],
                _block_diag(p[f"arel_{l}_{rel}"]),
                _block_diag(p[f"mrel_{l}_{rel}"]),
            )
            didx_g, sidx_g, didx_s = idx[rel]
            qe, kve = _sc_gather(q[dst], kvtab, didx_g, sidx_g)
            msgx = _edge_messages(qe, kve, _pad128(p[f"prel_{l}_{rel}"]))
            acc = _sc_scatter(msgx, didx_s, zeros_acc)
            msgn[dst] = _normalize(acc)

        for nt in ("author", "paper"):
            x[nt] = _out_proj(
                msgn[nt],
                p[f"Wa_{l}_{nt}"],
                p[f"bWa_{l}_{nt}"],
                _pad128(p[f"skip_{l}_{nt}"]),
                x[nt],
            )

    return x["author"], x["paper"]


# speedup vs baseline: 10.2598x; 1.0598x over previous
import functools
import jax
import jax.numpy as jnp
from jax import lax
from jax.experimental import pallas as pl
from jax.experimental.pallas import tpu as pltpu
from jax.experimental.pallas import tpu_sc as plsc

N = 10000
NP = 10112
HID = 512
H = 8
D = 64
E = 160000
EP = 163840
NC, NS = 2, 16
NW = NC * NS
TPW = EP // NW
CH = 128
NCH = TPW // CH
CH2 = 64
NCH2 = TPW // CH2
MW = 80
HP = H // 2
MW2 = 2 * MW
RSL = NP // NS

_mesh = plsc.VectorSubcoreMesh(
    core_axis_name="c", subcore_axis_name="s", num_cores=NC, num_subcores=NS
)


def _linear(x, w, b, act, br=1000):
    m, k = x.shape
    n = w.shape[1]

    def body(x_ref, w_ref, b_ref, o_ref):
        y = jnp.dot(x_ref[...], w_ref[...], preferred_element_type=jnp.float32)
        y = y + b_ref[...]
        o_ref[...] = act(y)

    return pl.pallas_call(
        body,
        grid=(m // br,),
        in_specs=[
            pl.BlockSpec((br, k), lambda i: (i, 0)),
            pl.BlockSpec((k, n), lambda i: (0, 0)),
            pl.BlockSpec((1, n), lambda i: (0, 0)),
        ],
        out_specs=pl.BlockSpec((br, n), lambda i: (i, 0)),
        out_shape=jax.ShapeDtypeStruct((m, n), jnp.float32),
    )(x, w, b.reshape(1, n))


def _block_diag(rel):
    bd = jnp.zeros((HID, HID), jnp.float32)
    for h in range(H):
        bd = bd.at[h * D:(h + 1) * D, h * D:(h + 1) * D].set(rel[h])
    return bd


def _kv_rel(k, v, bda, bdm, br=1000):
    def body(k_ref, v_ref, a_ref, m_ref, o_ref):
        kr = jnp.dot(k_ref[...], a_ref[...], preferred_element_type=jnp.float32)
        vr = jnp.dot(v_ref[...], m_ref[...], preferred_element_type=jnp.float32)
        o_ref[...] = jnp.concatenate([kr, vr], axis=1)

    return pl.pallas_call(
        body,
        grid=(N // br,),
        in_specs=[
            pl.BlockSpec((br, HID), lambda i: (i, 0)),
            pl.BlockSpec((br, HID), lambda i: (i, 0)),
            pl.BlockSpec((HID, HID), lambda i: (0, 0)),
            pl.BlockSpec((HID, HID), lambda i: (0, 0)),
        ],
        out_specs=pl.BlockSpec((br, 2 * HID), lambda i: (i, 0)),
        out_shape=jax.ShapeDtypeStruct((N, 2 * HID), jnp.float32),
    )(k, v, bda, bdm)


def _edge_messages(qe, kve, prel128, be=2048):
    def body(q_ref, kv_ref, p_ref, o_ref):
        i = pl.program_id(0)
        q = q_ref[...]
        k = kv_ref[:, :HID]
        v = kv_ref[:, HID:]
        rows = i * be + lax.broadcasted_iota(jnp.int32, (be, 1), 0)
        mask = rows < E
        lane = lax.broadcasted_iota(jnp.int32, (be, 16), 1)
        for h in range(H):
            sl = slice(h * D, (h + 1) * D)
            ah = jnp.sum(q[:, sl] * k[:, sl], axis=1, keepdims=True)
            ah = ah * (p_ref[0, h] * 0.125)
            exh = jnp.where(mask, jnp.exp(ah), 0.0)
            o_ref[h] = jnp.concatenate(
                [v[:, sl] * exh, jnp.where(lane == 0, exh, 0.0)], axis=1
            )

    return pl.pallas_call(
        body,
        grid=(EP // be,),
        in_specs=[
            pl.BlockSpec((be, HID), lambda i: (i, 0)),
            pl.BlockSpec((be, 2 * HID), lambda i: (i, 0)),
            pl.BlockSpec((1, 128), lambda i: (0, 0)),
        ],
        out_specs=pl.BlockSpec((H, be, MW), lambda i: (0, i, 0)),
        out_shape=jax.ShapeDtypeStruct((H, EP, MW), jnp.float32),
    )(qe, kve, prel128)


def _normalize(acc, br=1000):
    def body(a_ref, o_ref):
        a = a_ref[0] + a_ref[1]
        pieces = [
            a[h, :, 0:D] / (a[h, :, D:D + 1] + 1e-16) for h in range(H)
        ]
        o_ref[...] = jnp.concatenate(pieces, axis=1)

    return pl.pallas_call(
        body,
        grid=(N // br,),
        in_specs=[
            pl.BlockSpec((2, H, br, MW), lambda i: (0, 0, i, 0)),
        ],
        out_specs=pl.BlockSpec((br, HID), lambda i: (i, 0)),
        out_shape=jax.ShapeDtypeStruct((N, HID), jnp.float32),
    )(acc)


def _out_proj(msgn, wa, ba, skip128, xprev, br=1000):
    def body(m_ref, w_ref, b_ref, s_ref, x_ref, o_ref):
        o = jnp.dot(jax.nn.gelu(m_ref[...]), w_ref[...],
                    preferred_element_type=jnp.float32) + b_ref[...]
        beta = jax.nn.sigmoid(s_ref[0, 0])
        o_ref[...] = beta * o + (1.0 - beta) * x_ref[...]

    return pl.pallas_call(
        body,
        grid=(N // br,),
        in_specs=[
            pl.BlockSpec((br, HID), lambda i: (i, 0)),
            pl.BlockSpec((HID, HID), lambda i: (0, 0)),
            pl.BlockSpec((1, HID), lambda i: (0, 0)),
            pl.BlockSpec((1, 128), lambda i: (0, 0)),
            pl.BlockSpec((br, HID), lambda i: (i, 0)),
        ],
        out_specs=pl.BlockSpec((br, HID), lambda i: (i, 0)),
        out_shape=jax.ShapeDtypeStruct((N, HID), jnp.float32),
    )(msgn, wa, ba.reshape(1, HID), skip128, xprev)


@functools.partial(
    pl.kernel,
    mesh=_mesh,
    out_type=(
        jax.ShapeDtypeStruct((EP, HID), jnp.float32),
        jax.ShapeDtypeStruct((EP, 2 * HID), jnp.float32),
    ),
    scratch_types=[
        pltpu.VMEM((NCH2, CH2), jnp.int32),
        pltpu.VMEM((NCH2, CH2), jnp.int32),
        pltpu.VMEM((CH2, HID), jnp.float32),
        pltpu.VMEM((CH2, 2 * HID), jnp.float32),
        pltpu.SemaphoreType.DMA,
        pltpu.SemaphoreType.DMA,
        pltpu.SemaphoreType.DMA,
        pltpu.SemaphoreType.DMA,
    ],
)
def _sc_gather(qtab, kvtab, didx_hbm, sidx_hbm,
               qe, kve, didx_v, sidx_v, bq, bkv, semq, semk, ssq, ssk):
    c = lax.axis_index("c")
    s = lax.axis_index("s")
    wid = s * NC + c
    base = wid * TPW
    pltpu.sync_copy(didx_hbm.at[wid], didx_v)
    pltpu.sync_copy(sidx_hbm.at[wid], sidx_v)

    def chunk(j, carry):
        off = pl.multiple_of(base + j * CH2, CH2)
        gq = pltpu.async_copy(qtab.at[didx_v.at[j]], bq, semq)
        gk = pltpu.async_copy(kvtab.at[sidx_v.at[j]], bkv, semk)
        gq.wait()
        sq = pltpu.async_copy(bq, qe.at[pl.ds(off, CH2)], ssq)
        gk.wait()
        sk = pltpu.async_copy(bkv, kve.at[pl.ds(off, CH2)], ssk)
        sq.wait()
        sk.wait()
        return carry

    lax.fori_loop(0, NCH2, chunk, 0)


@functools.partial(
    pl.kernel,
    mesh=_mesh,
    out_type=jax.ShapeDtypeStruct((NC, H, NP, MW), jnp.float32),
    scratch_types=[
        pltpu.VMEM((NCH, CH), jnp.int32),
        pltpu.VMEM((CH, MW), jnp.float32),
        pltpu.VMEM((CH, MW), jnp.float32),
        pltpu.VMEM_SHARED((NP, MW), jnp.float32),
        pltpu.SemaphoreType.DMA,
        pltpu.SemaphoreType.DMA,
        pltpu.SemaphoreType.DMA,
        pltpu.SemaphoreType.DMA,
    ],
)
def _sc_scatter(msg_hbm, didx_hbm, zeros_hbm, out_hbm,
                didx_v, b0, b1, acc_sh, sl0, sl1, ss0, ss1):
    c = lax.axis_index("c")
    s = lax.axis_index("s")
    wid = s * NC + c
    base = wid * TPW
    pltpu.sync_copy(didx_hbm.at[wid], didx_v)
    myrows = pl.ds(s * RSL, RSL)
    for g in range(H):
        pltpu.sync_copy(zeros_hbm.at[myrows], acc_sh.at[myrows])
        plsc.subcore_barrier()

        def chunk(t, carry):
            j0 = t * 2
            j1 = j0 + 1
            off0 = pl.multiple_of(base + j0 * CH, CH)
            off1 = pl.multiple_of(base + j1 * CH, CH)
            l0 = pltpu.async_copy(msg_hbm.at[g].at[pl.ds(off0, CH)], b0, sl0)
            l1 = pltpu.async_copy(msg_hbm.at[g].at[pl.ds(off1, CH)], b1, sl1)
            l0.wait()
            s0 = pltpu.async_copy(b0, acc_sh.at[didx_v.at[j0]], ss0, add=True)
            l1.wait()
            s1 = pltpu.async_copy(b1, acc_sh.at[didx_v.at[j1]], ss1, add=True)
            s0.wait()
            s1.wait()
            return carry

        lax.fori_loop(0, NCH // 2, chunk, 0)
        plsc.subcore_barrier()
        pltpu.sync_copy(acc_sh.at[myrows], out_hbm.at[c].at[g].at[myrows])
        plsc.subcore_barrier()


def _prep_idx(ei):
    s = jnp.zeros((EP,), jnp.int32).at[:E].set(ei[0])
    d = jnp.zeros((EP,), jnp.int32).at[:E].set(ei[1])
    return (
        d.reshape(NW, NCH2, CH2),
        s.reshape(NW, NCH2, CH2),
        d.reshape(NW, NCH, CH),
    )


def _pad128(v):
    v = v.reshape(-1)
    return jnp.zeros((1, 128), jnp.float32).at[0, : v.shape[0]].set(v)


def kernel(x_author, x_paper, edge_index_writes, edge_index_rev_writes, params):
    p = params
    zeros_acc = jnp.zeros((NP, MW), jnp.float32)
    idx = {
        "writes": _prep_idx(edge_index_writes),
        "rev_writes": _prep_idx(edge_index_rev_writes),
    }
    edge_types = [
        ("author", "writes", "paper"),
        ("paper", "rev_writes", "author"),
    ]

    x = {
        "author": _linear(x_author, p["Win_author"], p["bin_author"], jax.nn.relu),
        "paper": _linear(x_paper, p["Win_paper"], p["bin_paper"], jax.nn.relu),
    }

    for l in range(2):
        q, k, v = {}, {}, {}
        for nt in ("author", "paper"):
            wqkv = jnp.concatenate(
                [p[f"Wq_{l}_{nt}"], p[f"Wk_{l}_{nt}"], p[f"Wv_{l}_{nt}"]], axis=1
            )
            bqkv = jnp.concatenate(
                [p[f"bWq_{l}_{nt}"], p[f"bWk_{l}_{nt}"], p[f"bWv_{l}_{nt}"]]
            )
            qkv = _linear(x[nt], wqkv, bqkv, lambda y: y)
            q[nt] = qkv[:, :HID]
            k[nt] = qkv[:, HID:2 * HID]
            v[nt] = qkv[:, 2 * HID:]

        msgn = {}
        for (src, rel, dst) in edge_types:
            kvtab = _kv_rel(
                k[src], v[src],
                _block_diag(p[f"arel_{l}_{rel}"]),
                _block_diag(p[f"mrel_{l}_{rel}"]),
            )
            didx_g, sidx_g, didx_s = idx[rel]
            qe, kve = _sc_gather(q[dst], kvtab, didx_g, sidx_g)
            msgx = _edge_messages(qe, kve, _pad128(p[f"prel_{l}_{rel}"]))
            acc = _sc_scatter(msgx, didx_s, zeros_acc)
            msgn[dst] = _normalize(acc)

        for nt in ("author", "paper"):
            x[nt] = _out_proj(
                msgn[nt],
                p[f"Wa_{l}_{nt}"],
                p[f"bWa_{l}_{nt}"],
                _pad128(p[f"skip_{l}_{nt}"]),
                x[nt],
            )

    return x["author"], x["paper"]


# speedup vs baseline: 10.6106x; 1.0342x over previous
import functools
import jax
import jax.numpy as jnp
from jax import lax
from jax.experimental import pallas as pl
from jax.experimental.pallas import tpu as pltpu
from jax.experimental.pallas import tpu_sc as plsc

N = 10000
NP = 10112
HID = 512
H = 8
D = 64
E = 160000
EP = 163840
NC, NS = 2, 16
NW = NC * NS
TPW = EP // NW
CH = 128
NCH = TPW // CH
CH2 = 32
NCH2 = TPW // CH2
MW = 80
HP = H // 2
MW2 = 2 * MW
RSL = NP // NS

_mesh = plsc.VectorSubcoreMesh(
    core_axis_name="c", subcore_axis_name="s", num_cores=NC, num_subcores=NS
)


def _linear(x, w, b, act, br=1000):
    m, k = x.shape
    n = w.shape[1]

    def body(x_ref, w_ref, b_ref, o_ref):
        y = jnp.dot(x_ref[...], w_ref[...], preferred_element_type=jnp.float32)
        y = y + b_ref[...]
        o_ref[...] = act(y)

    return pl.pallas_call(
        body,
        grid=(m // br,),
        in_specs=[
            pl.BlockSpec((br, k), lambda i: (i, 0)),
            pl.BlockSpec((k, n), lambda i: (0, 0)),
            pl.BlockSpec((1, n), lambda i: (0, 0)),
        ],
        out_specs=pl.BlockSpec((br, n), lambda i: (i, 0)),
        out_shape=jax.ShapeDtypeStruct((m, n), jnp.float32),
    )(x, w, b.reshape(1, n))


def _block_diag(rel):
    bd = jnp.zeros((HID, HID), jnp.float32)
    for h in range(H):
        bd = bd.at[h * D:(h + 1) * D, h * D:(h + 1) * D].set(rel[h])
    return bd


def _kv_rel(k, v, bda, bdm, br=1000):
    def body(k_ref, v_ref, a_ref, m_ref, o_ref):
        kr = jnp.dot(k_ref[...], a_ref[...], preferred_element_type=jnp.float32)
        vr = jnp.dot(v_ref[...], m_ref[...], preferred_element_type=jnp.float32)
        o_ref[...] = jnp.concatenate([kr, vr], axis=1)

    return pl.pallas_call(
        body,
        grid=(N // br,),
        in_specs=[
            pl.BlockSpec((br, HID), lambda i: (i, 0)),
            pl.BlockSpec((br, HID), lambda i: (i, 0)),
            pl.BlockSpec((HID, HID), lambda i: (0, 0)),
            pl.BlockSpec((HID, HID), lambda i: (0, 0)),
        ],
        out_specs=pl.BlockSpec((br, 2 * HID), lambda i: (i, 0)),
        out_shape=jax.ShapeDtypeStruct((N, 2 * HID), jnp.float32),
    )(k, v, bda, bdm)


def _edge_messages(qe, kve, prel128, be=2048):
    def body(q_ref, kv_ref, p_ref, o_ref):
        i = pl.program_id(0)
        q = q_ref[...]
        k = kv_ref[:, :HID]
        v = kv_ref[:, HID:]
        rows = i * be + lax.broadcasted_iota(jnp.int32, (be, 1), 0)
        mask = rows < E
        lane = lax.broadcasted_iota(jnp.int32, (be, 16), 1)
        for h in range(H):
            sl = slice(h * D, (h + 1) * D)
            ah = jnp.sum(q[:, sl] * k[:, sl], axis=1, keepdims=True)
            ah = ah * (p_ref[0, h] * 0.125)
            exh = jnp.where(mask, jnp.exp(ah), 0.0)
            o_ref[h] = jnp.concatenate(
                [v[:, sl] * exh, jnp.where(lane == 0, exh, 0.0)], axis=1
            )

    return pl.pallas_call(
        body,
        grid=(EP // be,),
        in_specs=[
            pl.BlockSpec((be, HID), lambda i: (i, 0)),
            pl.BlockSpec((be, 2 * HID), lambda i: (i, 0)),
            pl.BlockSpec((1, 128), lambda i: (0, 0)),
        ],
        out_specs=pl.BlockSpec((H, be, MW), lambda i: (0, i, 0)),
        out_shape=jax.ShapeDtypeStruct((H, EP, MW), jnp.float32),
    )(qe, kve, prel128)


def _normalize(acc, br=1000):
    def body(a_ref, o_ref):
        a = a_ref[0] + a_ref[1]
        pieces = [
            a[h, :, 0:D] / (a[h, :, D:D + 1] + 1e-16) for h in range(H)
        ]
        o_ref[...] = jnp.concatenate(pieces, axis=1)

    return pl.pallas_call(
        body,
        grid=(N // br,),
        in_specs=[
            pl.BlockSpec((2, H, br, MW), lambda i: (0, 0, i, 0)),
        ],
        out_specs=pl.BlockSpec((br, HID), lambda i: (i, 0)),
        out_shape=jax.ShapeDtypeStruct((N, HID), jnp.float32),
    )(acc)


def _out_proj(msgn, wa, ba, skip128, xprev, br=1000):
    def body(m_ref, w_ref, b_ref, s_ref, x_ref, o_ref):
        o = jnp.dot(jax.nn.gelu(m_ref[...]), w_ref[...],
                    preferred_element_type=jnp.float32) + b_ref[...]
        beta = jax.nn.sigmoid(s_ref[0, 0])
        o_ref[...] = beta * o + (1.0 - beta) * x_ref[...]

    return pl.pallas_call(
        body,
        grid=(N // br,),
        in_specs=[
            pl.BlockSpec((br, HID), lambda i: (i, 0)),
            pl.BlockSpec((HID, HID), lambda i: (0, 0)),
            pl.BlockSpec((1, HID), lambda i: (0, 0)),
            pl.BlockSpec((1, 128), lambda i: (0, 0)),
            pl.BlockSpec((br, HID), lambda i: (i, 0)),
        ],
        out_specs=pl.BlockSpec((br, HID), lambda i: (i, 0)),
        out_shape=jax.ShapeDtypeStruct((N, HID), jnp.float32),
    )(msgn, wa, ba.reshape(1, HID), skip128, xprev)


@functools.partial(
    pl.kernel,
    mesh=_mesh,
    out_type=(
        jax.ShapeDtypeStruct((EP, HID), jnp.float32),
        jax.ShapeDtypeStruct((EP, 2 * HID), jnp.float32),
    ),
    scratch_types=[
        pltpu.VMEM((NCH, CH), jnp.int32),
        pltpu.VMEM((NCH, CH), jnp.int32),
        pltpu.VMEM((CH2, HID), jnp.float32),
        pltpu.VMEM((CH2, HID), jnp.float32),
        pltpu.VMEM((CH2, 2 * HID), jnp.float32),
        pltpu.VMEM((CH2, 2 * HID), jnp.float32),
        pltpu.SemaphoreType.DMA,
        pltpu.SemaphoreType.DMA,
        pltpu.SemaphoreType.DMA,
        pltpu.SemaphoreType.DMA,
        pltpu.SemaphoreType.DMA,
        pltpu.SemaphoreType.DMA,
        pltpu.SemaphoreType.DMA,
        pltpu.SemaphoreType.DMA,
    ],
)
def _sc_gather(qtab, kvtab, didx_hbm, sidx_hbm, qe, kve,
               didx_v, sidx_v, bq0, bq1, bkv0, bkv1,
               s1, s2, s3, s4, s5, s6, s7, s8):
    c = lax.axis_index("c")
    s = lax.axis_index("s")
    wid = s * NC + c
    base = wid * TPW
    pltpu.sync_copy(didx_hbm.at[wid], didx_v)
    pltpu.sync_copy(sidx_hbm.at[wid], sidx_v)
    bq = (bq0, bq1)
    bkv = (bkv0, bkv1)
    gsem = (s1, s2)
    ksem = (s3, s4)
    tqs = (s5, s6)
    tks = (s7, s8)

    def chunk(r, carry):
        stores = [None, None]
        for u in range(CH // CH2):
            b = u % 2
            if stores[b] is not None:
                stores[b][0].wait()
                stores[b][1].wait()
            isl = pl.ds(u * CH2, CH2)
            off = pl.multiple_of(base + r * CH + u * CH2, CH2)
            gq = pltpu.async_copy(qtab.at[didx_v.at[r, isl]], bq[b], gsem[b])
            gk = pltpu.async_copy(kvtab.at[sidx_v.at[r, isl]], bkv[b], ksem[b])
            gq.wait()
            tq = pltpu.async_copy(bq[b], qe.at[pl.ds(off, CH2)], tqs[b])
            gk.wait()
            tk = pltpu.async_copy(bkv[b], kve.at[pl.ds(off, CH2)], tks[b])
            stores[b] = (tq, tk)
        for st in stores:
            st[0].wait()
            st[1].wait()
        return carry

    lax.fori_loop(0, NCH, chunk, 0)


@functools.partial(
    pl.kernel,
    mesh=_mesh,
    out_type=jax.ShapeDtypeStruct((NC, H, NP, MW), jnp.float32),
    scratch_types=[
        pltpu.VMEM((NCH, CH), jnp.int32),
        pltpu.VMEM((CH, MW), jnp.float32),
        pltpu.VMEM((CH, MW), jnp.float32),
        pltpu.VMEM_SHARED((NP, MW), jnp.float32),
        pltpu.SemaphoreType.DMA,
        pltpu.SemaphoreType.DMA,
        pltpu.SemaphoreType.DMA,
        pltpu.SemaphoreType.DMA,
    ],
)
def _sc_scatter(msg_hbm, didx_hbm, zeros_hbm, out_hbm,
                didx_v, b0, b1, acc_sh, sl0, sl1, ss0, ss1):
    c = lax.axis_index("c")
    s = lax.axis_index("s")
    wid = s * NC + c
    base = wid * TPW
    pltpu.sync_copy(didx_hbm.at[wid], didx_v)
    myrows = pl.ds(s * RSL, RSL)
    for g in range(H):
        pltpu.sync_copy(zeros_hbm.at[myrows], acc_sh.at[myrows])
        plsc.subcore_barrier()

        def chunk(t, carry):
            j0 = t * 2
            j1 = j0 + 1
            off0 = pl.multiple_of(base + j0 * CH, CH)
            off1 = pl.multiple_of(base + j1 * CH, CH)
            l0 = pltpu.async_copy(msg_hbm.at[g].at[pl.ds(off0, CH)], b0, sl0)
            l1 = pltpu.async_copy(msg_hbm.at[g].at[pl.ds(off1, CH)], b1, sl1)
            l0.wait()
            s0 = pltpu.async_copy(b0, acc_sh.at[didx_v.at[j0]], ss0, add=True)
            l1.wait()
            s1 = pltpu.async_copy(b1, acc_sh.at[didx_v.at[j1]], ss1, add=True)
            s0.wait()
            s1.wait()
            return carry

        lax.fori_loop(0, NCH // 2, chunk, 0)
        plsc.subcore_barrier()
        pltpu.sync_copy(acc_sh.at[myrows], out_hbm.at[c].at[g].at[myrows])
        plsc.subcore_barrier()


def _prep_idx(ei):
    s = jnp.zeros((EP,), jnp.int32).at[:E].set(ei[0])
    d = jnp.zeros((EP,), jnp.int32).at[:E].set(ei[1])
    return d.reshape(NW, NCH, CH), s.reshape(NW, NCH, CH)


def _pad128(v):
    v = v.reshape(-1)
    return jnp.zeros((1, 128), jnp.float32).at[0, : v.shape[0]].set(v)


def kernel(x_author, x_paper, edge_index_writes, edge_index_rev_writes, params):
    p = params
    zeros_acc = jnp.zeros((NP, MW), jnp.float32)
    idx = {
        "writes": _prep_idx(edge_index_writes),
        "rev_writes": _prep_idx(edge_index_rev_writes),
    }
    edge_types = [
        ("author", "writes", "paper"),
        ("paper", "rev_writes", "author"),
    ]

    x = {
        "author": _linear(x_author, p["Win_author"], p["bin_author"], jax.nn.relu),
        "paper": _linear(x_paper, p["Win_paper"], p["bin_paper"], jax.nn.relu),
    }

    for l in range(2):
        q, k, v = {}, {}, {}
        for nt in ("author", "paper"):
            wqkv = jnp.concatenate(
                [p[f"Wq_{l}_{nt}"], p[f"Wk_{l}_{nt}"], p[f"Wv_{l}_{nt}"]], axis=1
            )
            bqkv = jnp.concatenate(
                [p[f"bWq_{l}_{nt}"], p[f"bWk_{l}_{nt}"], p[f"bWv_{l}_{nt}"]]
            )
            qkv = _linear(x[nt], wqkv, bqkv, lambda y: y)
            q[nt] = qkv[:, :HID]
            k[nt] = qkv[:, HID:2 * HID]
            v[nt] = qkv[:, 2 * HID:]

        msgn = {}
        for (src, rel, dst) in edge_types:
            kvtab = _kv_rel(
                k[src], v[src],
                _block_diag(p[f"arel_{l}_{rel}"]),
                _block_diag(p[f"mrel_{l}_{rel}"]),
            )
            didx3, sidx3 = idx[rel]
            qe, kve = _sc_gather(q[dst], kvtab, didx3, sidx3)
            msgx = _edge_messages(qe, kve, _pad128(p[f"prel_{l}_{rel}"]))
            acc = _sc_scatter(msgx, didx3, zeros_acc)
            msgn[dst] = _normalize(acc)

        for nt in ("author", "paper"):
            x[nt] = _out_proj(
                msgn[nt],
                p[f"Wa_{l}_{nt}"],
                p[f"bWa_{l}_{nt}"],
                _pad128(p[f"skip_{l}_{nt}"]),
                x[nt],
            )

    return x["author"], x["paper"]
